# Initial kernel scaffold; baseline (speedup 1.0000x reference)
import jax
import jax.numpy as jnp
from jax.experimental import pallas as pl


def kernel(x, edge_index, W1, b1, W2, b2, W3, b3, W4, b4, W5, b5, g1, be1, rm1, rv1, g2, be2, rm2, rv2, g3, be3, rm3, rv3, g4, be4, rm4, rv4):
    raise NotImplementedError("write your pallas kernel here")


# speedup vs baseline: 6.8830x; 6.8830x over previous
import functools

import jax
import jax.numpy as jnp
from jax import lax
from jax.experimental import pallas as pl
from jax.experimental.pallas import tpu as pltpu
from jax.experimental.pallas import tpu_sc as plsc

N_NODES = 10000
N_EDGES = 320000
FEAT = 128
BN_EPS_K = 1e-5

NC = 2
NS = 16
NW = NC * NS
EDGES_PER_W = N_EDGES // NW
CHUNK_E = 100
CHUNKS = EDGES_PER_W // CHUNK_E
N_PAD = 10240
ROWS_PER_TILE = N_PAD // NS


def _sc_agg_body(h_hbm, src_hbm, dst_hbm, zero_hbm, out_hbm,
                 src_v, dst_v, rows_v, sem_g, acc_sh):
    cid = lax.axis_index("c")
    sid = lax.axis_index("s")
    wid = sid * NC + cid

    pltpu.sync_copy(zero_hbm, acc_sh.at[pl.ds(sid * ROWS_PER_TILE, ROWS_PER_TILE)])
    pltpu.sync_copy(src_hbm.at[wid], src_v)
    pltpu.sync_copy(dst_hbm.at[wid], dst_v)
    plsc.subcore_barrier()

    def loop_body(j, _):
        pltpu.async_copy(h_hbm.at[src_v.at[j]], rows_v, sem_g).wait()
        pltpu.sync_copy(rows_v, acc_sh.at[dst_v.at[j]], add=True)
        return 0

    lax.fori_loop(0, CHUNKS, loop_body, 0)
    plsc.subcore_barrier()

    r0 = sid * ROWS_PER_TILE
    pltpu.sync_copy(acc_sh.at[pl.ds(r0, ROWS_PER_TILE)],
                    out_hbm.at[cid].at[pl.ds(r0, ROWS_PER_TILE)])


@jax.jit
def _sc_agg(h, src_r, dst_r, zero_rows):
    mesh = plsc.VectorSubcoreMesh(core_axis_name="c", subcore_axis_name="s")
    fn = pl.kernel(
        _sc_agg_body,
        out_type=jax.ShapeDtypeStruct((NC, N_PAD, FEAT), jnp.float32),
        mesh=mesh,
        scratch_types=[
            pltpu.VMEM((CHUNKS, CHUNK_E), jnp.int32),
            pltpu.VMEM((CHUNKS, CHUNK_E), jnp.int32),
            pltpu.VMEM((CHUNK_E, FEAT), jnp.float32),
            pltpu.SemaphoreType.DMA,
            pltpu.VMEM_SHARED((N_PAD, FEAT), jnp.float32),
        ],
    )
    return fn(h, src_r, dst_r, zero_rows)


M_BLK = 1000
GRID_M = N_NODES // M_BLK


def _mlp2_body(x_ref, p0_ref, p1_ref, w1_ref, b1_ref, w2_ref, b2_ref, o_ref):
    h = x_ref[...] + p0_ref[...] + p1_ref[...]
    h = jnp.maximum(jnp.dot(h, w1_ref[...], preferred_element_type=jnp.float32)
                    + b1_ref[...], 0.0)
    h = jnp.maximum(jnp.dot(h, w2_ref[...], preferred_element_type=jnp.float32)
                    + b2_ref[...], 0.0)
    o_ref[...] = h


def _mlp3_body(x_ref, p0_ref, p1_ref, w1_ref, b1_ref, w2_ref, b2_ref,
               w3_ref, b3_ref, o_ref):
    h = x_ref[...] + p0_ref[...] + p1_ref[...]
    h = jnp.maximum(jnp.dot(h, w1_ref[...], preferred_element_type=jnp.float32)
                    + b1_ref[...], 0.0)
    h = jnp.maximum(jnp.dot(h, w2_ref[...], preferred_element_type=jnp.float32)
                    + b2_ref[...], 0.0)
    o_ref[...] = jnp.dot(h, w3_ref[...], preferred_element_type=jnp.float32) + b3_ref[...]


def _row_spec():
    return pl.BlockSpec((M_BLK, FEAT), lambda i: (i, 0))


def _full_spec(shape):
    return pl.BlockSpec(shape, lambda i: tuple(0 for _ in shape))


@jax.jit
def _tc_mlp2(x, p0, p1, w1, b1, w2, b2):
    return pl.pallas_call(
        _mlp2_body,
        out_shape=jax.ShapeDtypeStruct((N_NODES, FEAT), jnp.float32),
        grid=(GRID_M,),
        in_specs=[_row_spec(), _row_spec(), _row_spec(),
                  _full_spec((FEAT, FEAT)), _full_spec((1, FEAT)),
                  _full_spec((FEAT, FEAT)), _full_spec((1, FEAT))],
        out_specs=_row_spec(),
    )(x, p0, p1, w1, b1, w2, b2)


@jax.jit
def _tc_mlp3(x, p0, p1, w1, b1, w2, b2, w3, b3):
    return pl.pallas_call(
        _mlp3_body,
        out_shape=jax.ShapeDtypeStruct((N_NODES, FEAT), jnp.float32),
        grid=(GRID_M,),
        in_specs=[_row_spec(), _row_spec(), _row_spec(),
                  _full_spec((FEAT, FEAT)), _full_spec((1, FEAT)),
                  _full_spec((FEAT, FEAT)), _full_spec((1, FEAT)),
                  _full_spec((FEAT, FEAT)), _full_spec((1, FEAT))],
        out_specs=_row_spec(),
    )(x, p0, p1, w1, b1, w2, b2, w3, b3)


def _fold_bn(W, b, g, be, rm, rv):
    s = g / jnp.sqrt(rv + BN_EPS_K)
    wt = W.T * s[None, :]
    bf = ((b - rm) * s + be)[None, :]
    return wt, bf


def kernel(x, edge_index, W1, b1, W2, b2, W3, b3, W4, b4, W5, b5,
           g1, be1, rm1, rv1, g2, be2, rm2, rv2,
           g3, be3, rm3, rv3, g4, be4, rm4, rv4):
    src_r = edge_index[0].reshape(NW, CHUNKS, CHUNK_E)
    dst_r = edge_index[1].reshape(NW, CHUNKS, CHUNK_E)
    zero_rows = jnp.zeros((ROWS_PER_TILE, FEAT), jnp.float32)

    w1t, b1f = _fold_bn(W1, b1, g1, be1, rm1, rv1)
    w2t, b2f = _fold_bn(W2, b2, g2, be2, rm2, rv2)
    w3t, b3f = _fold_bn(W3, b3, g3, be3, rm3, rv3)
    w4t, b4f = _fold_bn(W4, b4, g4, be4, rm4, rv4)
    w5t, b5f = W5.T, b5[None, :]

    p = _sc_agg(x, src_r, dst_r, zero_rows)
    h = _tc_mlp2(x, p[0, :N_NODES], p[1, :N_NODES], w1t, b1f, w2t, b2f)
    q = _sc_agg(h, src_r, dst_r, zero_rows)
    out = _tc_mlp3(h, q[0, :N_NODES], q[1, :N_NODES], w3t, b3f, w4t, b4f, w5t, b5f)
    return out
